# baseline (speedup 1.0000x reference)
import functools

import jax
import jax.numpy as jnp
from jax.experimental import pallas as pl
from jax.experimental.pallas import tpu as pltpu

_EPS = 1e-12


def _power_iter_kernel(w_ref, u_ref, inv_sig_ref, u_out_ref, v_out_ref):
    w = w_ref[...]
    u = u_ref[...]
    vr = jnp.dot(u, w, preferred_element_type=jnp.float32)
    v = vr / (jnp.sqrt(jnp.sum(vr * vr)) + _EPS)
    wv = jax.lax.dot_general(
        v, w, dimension_numbers=(((1,), (1,)), ((), ())),
        preferred_element_type=jnp.float32)
    s2 = jnp.sum(wv * wv)
    nrm = jnp.sqrt(s2) + _EPS
    u_out_ref[...] = wv / nrm
    v_out_ref[...] = v
    inv_sig_ref[...] = jnp.full((1, 1), nrm / s2, jnp.float32)


def _conv_body(inv_sig_ref, x_ref, w_ref, b_ref, o_ref, zb_ref, cat_ref,
               *, taps, C, W, HW, halo):
    zb_ref[:, :halo] = jnp.zeros((C, halo), jnp.bfloat16)
    zb_ref[:, halo + HW:] = jnp.zeros((C, zb_ref.shape[1] - halo - HW),
                                      jnp.bfloat16)
    zb_ref[:, halo:halo + HW] = x_ref[0].astype(jnp.bfloat16).reshape(C, HW)

    col = jax.lax.broadcasted_iota(jnp.int32, (1, HW), 1) % W
    mask_l = (col > 0).astype(jnp.bfloat16)
    mask_r = (col < W - 1).astype(jnp.bfloat16)

    for t, (dh, dw) in enumerate(taps):
        off = halo + (dh - 1) * W + (dw - 1)
        xs = zb_ref[:, off:off + HW]
        if dw == 0:
            xs = xs * mask_l
        elif dw == 2:
            xs = xs * mask_r
        cat_ref[t * C:(t + 1) * C, :] = xs

    acc = jnp.dot(w_ref[...], cat_ref[...],
                  preferred_element_type=jnp.float32)
    res = acc * inv_sig_ref[0, 0] + b_ref[...]
    o_ref[0] = res.reshape(o_ref.shape[1:])


def kernel(x, w_bar, bias, u):
    N, C, H, W = x.shape
    O, Cw, kh, kw = w_bar.shape
    K = C * kh * kw
    HW = H * W
    T = kh * kw

    w_mat = w_bar.reshape(O, K).astype(jnp.float32)
    inv_sig, u_new, v_new = pl.pallas_call(
        _power_iter_kernel,
        out_shape=(
            jax.ShapeDtypeStruct((1, 1), jnp.float32),
            jax.ShapeDtypeStruct((1, O), jnp.float32),
            jax.ShapeDtypeStruct((1, K), jnp.float32),
        ),
    )(w_mat, u.reshape(1, O).astype(jnp.float32))

    w_cat = jnp.transpose(w_bar, (0, 2, 3, 1)).reshape(O, T * C)
    w_cat = w_cat.astype(jnp.bfloat16)

    halo = W + 1
    padw = ((HW + 2 * halo + 127) // 128) * 128
    taps = tuple((dh, dw) for dh in range(kh) for dw in range(kw))

    body = functools.partial(_conv_body, taps=taps, C=C, W=W, HW=HW,
                             halo=halo)
    y = pl.pallas_call(
        body,
        out_shape=jax.ShapeDtypeStruct((N, O, H, W), jnp.float32),
        grid=(N,),
        in_specs=[
            pl.BlockSpec((1, 1), lambda n: (0, 0)),
            pl.BlockSpec((1, C, H, W), lambda n: (n, 0, 0, 0)),
            pl.BlockSpec((O, T * C), lambda n: (0, 0)),
            pl.BlockSpec((O, 1), lambda n: (0, 0)),
        ],
        out_specs=pl.BlockSpec((1, O, H, W), lambda n: (n, 0, 0, 0)),
        scratch_shapes=[
            pltpu.VMEM((C, padw), jnp.bfloat16),
            pltpu.VMEM((T * C, HW), jnp.bfloat16),
        ],
        compiler_params=pltpu.CompilerParams(
            dimension_semantics=("parallel",)),
    )(inv_sig, x, w_cat, bias.reshape(O, 1).astype(jnp.float32))

    return (y, u_new.reshape(O), v_new.reshape(K))


# speedup vs baseline: 2.3479x; 2.3479x over previous
import functools

import jax
import jax.numpy as jnp
from jax.experimental import pallas as pl
from jax.experimental.pallas import tpu as pltpu

_EPS = 1e-12


def _power_iter_kernel(w_ref, u_ref, inv_sig_ref, u_out_ref, v_out_ref):
    w = w_ref[...]
    u = u_ref[...]
    vr = jnp.dot(u, w, preferred_element_type=jnp.float32)
    v = vr / (jnp.sqrt(jnp.sum(vr * vr)) + _EPS)
    wv = jax.lax.dot_general(
        v, w, dimension_numbers=(((1,), (1,)), ((), ())),
        preferred_element_type=jnp.float32)
    s2 = jnp.sum(wv * wv)
    nrm = jnp.sqrt(s2) + _EPS
    u_out_ref[...] = wv / nrm
    v_out_ref[...] = v
    inv_sig_ref[...] = jnp.full((1, 1), nrm / s2, jnp.float32)


def _conv_body(inv_sig_ref, x_ref, w_ref, b_ref, o_ref, zb_ref,
               *, taps, C, W, HW, halo):
    zb_ref[:, :halo] = jnp.zeros((C, halo), jnp.bfloat16)
    zb_ref[:, halo + HW:] = jnp.zeros((C, zb_ref.shape[1] - halo - HW),
                                      jnp.bfloat16)
    zb_ref[:, halo:halo + HW] = x_ref[0].astype(jnp.bfloat16)

    col = jax.lax.broadcasted_iota(jnp.int32, (1, HW), 1) % W
    mask_l = (col > 0).astype(jnp.bfloat16)
    mask_r = (col < W - 1).astype(jnp.bfloat16)

    acc = None
    for t, (dh, dw) in enumerate(taps):
        off = halo + (dh - 1) * W + (dw - 1)
        xs = zb_ref[:, off:off + HW]
        if dw == 0:
            xs = xs * mask_l
        elif dw == 2:
            xs = xs * mask_r
        d = jax.lax.dot_general(
            w_ref[t], xs, dimension_numbers=(((1,), (0,)), ((), ())),
            preferred_element_type=jnp.float32)
        acc = d if acc is None else acc + d
    o_ref[0] = acc * inv_sig_ref[0, 0] + b_ref[...]


def kernel(x, w_bar, bias, u):
    N, C, H, W = x.shape
    O, Cw, kh, kw = w_bar.shape
    K = C * kh * kw
    HW = H * W
    T = kh * kw

    w_mat = w_bar.reshape(O, K).astype(jnp.float32)
    inv_sig, u_new, v_new = pl.pallas_call(
        _power_iter_kernel,
        out_shape=(
            jax.ShapeDtypeStruct((1, 1), jnp.float32),
            jax.ShapeDtypeStruct((1, O), jnp.float32),
            jax.ShapeDtypeStruct((1, K), jnp.float32),
        ),
    )(w_mat, u.reshape(1, O).astype(jnp.float32))

    w_taps = jnp.transpose(w_bar, (2, 3, 0, 1)).reshape(T, O, C)
    w_taps = w_taps.astype(jnp.bfloat16)

    halo = W + 1
    padw = ((HW + 2 * halo + 127) // 128) * 128
    taps = tuple((dh, dw) for dh in range(kh) for dw in range(kw))

    body = functools.partial(_conv_body, taps=taps, C=C, W=W, HW=HW,
                             halo=halo)
    y_flat = pl.pallas_call(
        body,
        out_shape=jax.ShapeDtypeStruct((N, O, HW), jnp.float32),
        grid=(N,),
        in_specs=[
            pl.BlockSpec((1, 1), lambda n: (0, 0)),
            pl.BlockSpec((1, C, HW), lambda n: (n, 0, 0)),
            pl.BlockSpec((T, O, C), lambda n: (0, 0, 0)),
            pl.BlockSpec((O, 1), lambda n: (0, 0)),
        ],
        out_specs=pl.BlockSpec((1, O, HW), lambda n: (n, 0, 0)),
        scratch_shapes=[
            pltpu.VMEM((C, padw), jnp.bfloat16),
        ],
        compiler_params=pltpu.CompilerParams(
            dimension_semantics=("parallel",)),
    )(inv_sig, x.reshape(N, C, HW), w_taps, bias.reshape(O, 1).astype(jnp.float32))

    return (y_flat.reshape(N, O, H, W), u_new.reshape(O), v_new.reshape(K))


# speedup vs baseline: 2.5276x; 1.0765x over previous
import functools

import jax
import jax.numpy as jnp
from jax.experimental import pallas as pl
from jax.experimental.pallas import tpu as pltpu

_EPS = 1e-12


def _power_iter_kernel(w_ref, u_ref, inv_sig_ref, u_out_ref, v_out_ref):
    w = w_ref[...]
    u = u_ref[...]
    vr = jnp.dot(u, w, preferred_element_type=jnp.float32)
    v = vr / (jnp.sqrt(jnp.sum(vr * vr)) + _EPS)
    wv = jax.lax.dot_general(
        v, w, dimension_numbers=(((1,), (1,)), ((), ())),
        preferred_element_type=jnp.float32)
    s2 = jnp.sum(wv * wv)
    nrm = jnp.sqrt(s2) + _EPS
    u_out_ref[...] = wv / nrm
    v_out_ref[...] = v
    inv_sig_ref[...] = jnp.full((1, 1), nrm / s2, jnp.float32)


def _conv_body(inv_sig_ref, x_ref, w_ref, b_ref, o_ref, zb_ref,
               *, taps, C, W, HW, halo):
    col = jax.lax.broadcasted_iota(jnp.int32, (1, HW), 1) % W
    mask_l = (col > 0).astype(jnp.bfloat16)
    mask_r = (col < W - 1).astype(jnp.bfloat16)
    inv_sig = inv_sig_ref[0, 0]
    bias = b_ref[...]

    zb_ref[:, :halo] = jnp.zeros((C, halo), jnp.bfloat16)
    zb_ref[:, halo + HW:] = jnp.zeros((C, zb_ref.shape[1] - halo - HW),
                                      jnp.bfloat16)

    for b in range(x_ref.shape[0]):
        zb_ref[:, halo:halo + HW] = x_ref[b]
        acc = None
        for t, (dh, dw) in enumerate(taps):
            off = halo + (dh - 1) * W + (dw - 1)
            xs = zb_ref[:, off:off + HW]
            if dw == 0:
                xs = xs * mask_l
            elif dw == 2:
                xs = xs * mask_r
            d = jax.lax.dot_general(
                w_ref[t], xs, dimension_numbers=(((1,), (0,)), ((), ())),
                preferred_element_type=jnp.float32)
            acc = d if acc is None else acc + d
        o_ref[b] = acc * inv_sig + bias


def kernel(x, w_bar, bias, u):
    N, C, H, W = x.shape
    O, Cw, kh, kw = w_bar.shape
    K = C * kh * kw
    HW = H * W
    T = kh * kw

    w_mat = w_bar.reshape(O, K).astype(jnp.float32)
    inv_sig, u_new, v_new = pl.pallas_call(
        _power_iter_kernel,
        out_shape=(
            jax.ShapeDtypeStruct((1, 1), jnp.float32),
            jax.ShapeDtypeStruct((1, O), jnp.float32),
            jax.ShapeDtypeStruct((1, K), jnp.float32),
        ),
    )(w_mat, u.reshape(1, O).astype(jnp.float32))

    w_taps = jnp.transpose(w_bar, (2, 3, 0, 1)).reshape(T, O, C)
    w_taps = w_taps.astype(jnp.bfloat16)

    halo = W + 1
    padw = ((HW + 2 * halo + 127) // 128) * 128
    taps = tuple((dh, dw) for dh in range(kh) for dw in range(kw))

    body = functools.partial(_conv_body, taps=taps, C=C, W=W, HW=HW,
                             halo=halo)
    B = 2
    xb = x.reshape(N, C, HW).astype(jnp.bfloat16)
    y_flat = pl.pallas_call(
        body,
        out_shape=jax.ShapeDtypeStruct((N, O, HW), jnp.float32),
        grid=(N // B,),
        in_specs=[
            pl.BlockSpec((1, 1), lambda n: (0, 0)),
            pl.BlockSpec((B, C, HW), lambda n: (n, 0, 0)),
            pl.BlockSpec((T, O, C), lambda n: (0, 0, 0)),
            pl.BlockSpec((O, 1), lambda n: (0, 0)),
        ],
        out_specs=pl.BlockSpec((B, O, HW), lambda n: (n, 0, 0)),
        scratch_shapes=[
            pltpu.VMEM((C, padw), jnp.bfloat16),
        ],
        compiler_params=pltpu.CompilerParams(
            dimension_semantics=("parallel",)),
    )(inv_sig, xb, w_taps, bias.reshape(O, 1).astype(jnp.float32))

    return (y_flat.reshape(N, O, H, W), u_new.reshape(O), v_new.reshape(K))


# speedup vs baseline: 2.6450x; 1.0464x over previous
import functools

import jax
import jax.numpy as jnp
from jax.experimental import pallas as pl
from jax.experimental.pallas import tpu as pltpu

_EPS = 1e-12


def _power_iter_kernel(w_ref, u_ref, inv_sig_ref, u_out_ref, v_out_ref):
    w = w_ref[...]
    u = u_ref[...]
    vr = jnp.dot(u, w, preferred_element_type=jnp.float32)
    v = vr / (jnp.sqrt(jnp.sum(vr * vr)) + _EPS)
    wv = jax.lax.dot_general(
        v, w, dimension_numbers=(((1,), (1,)), ((), ())),
        preferred_element_type=jnp.float32)
    s2 = jnp.sum(wv * wv)
    nrm = jnp.sqrt(s2) + _EPS
    u_out_ref[...] = wv / nrm
    v_out_ref[...] = v
    inv_sig_ref[...] = jnp.full((1, 1), nrm / s2, jnp.float32)


def _conv_body(inv_sig_ref, x_ref, w_ref, b_ref, o_ref, zb_ref,
               *, taps, C, W, HW, halo):
    col = jax.lax.broadcasted_iota(jnp.int32, (1, HW), 1) % W
    mask_l = (col > 0).astype(jnp.bfloat16)
    mask_r = (col < W - 1).astype(jnp.bfloat16)
    inv_sig = inv_sig_ref[0, 0]
    bias = b_ref[...]

    zb_ref[:, :halo] = jnp.zeros((C, halo), jnp.bfloat16)
    zb_ref[:, halo + HW:] = jnp.zeros((C, zb_ref.shape[1] - halo - HW),
                                      jnp.bfloat16)

    for b in range(x_ref.shape[0]):
        zb_ref[:, halo:halo + HW] = x_ref[b]
        acc = None
        for t, (dh, dw) in enumerate(taps):
            off = halo + (dh - 1) * W + (dw - 1)
            xs = zb_ref[:, off:off + HW]
            if dw == 0:
                xs = xs * mask_l
            elif dw == 2:
                xs = xs * mask_r
            d = jax.lax.dot_general(
                w_ref[t], xs, dimension_numbers=(((1,), (0,)), ((), ())),
                preferred_element_type=jnp.float32)
            acc = d if acc is None else acc + d
        o_ref[b] = acc * inv_sig + bias


def kernel(x, w_bar, bias, u):
    N, C, H, W = x.shape
    O, Cw, kh, kw = w_bar.shape
    K = C * kh * kw
    HW = H * W
    T = kh * kw

    w_mat = w_bar.reshape(O, K).astype(jnp.float32)
    inv_sig, u_new, v_new = pl.pallas_call(
        _power_iter_kernel,
        out_shape=(
            jax.ShapeDtypeStruct((1, 1), jnp.float32),
            jax.ShapeDtypeStruct((1, O), jnp.float32),
            jax.ShapeDtypeStruct((1, K), jnp.float32),
        ),
    )(w_mat, u.reshape(1, O).astype(jnp.float32))

    w_taps = jnp.transpose(w_bar, (2, 3, 0, 1)).reshape(T, O, C)
    w_taps = w_taps.astype(jnp.bfloat16)

    halo = W + 1
    padw = ((HW + 2 * halo + 127) // 128) * 128
    taps = tuple((dh, dw) for dh in range(kh) for dw in range(kw))

    body = functools.partial(_conv_body, taps=taps, C=C, W=W, HW=HW,
                             halo=halo)
    B = 4
    xb = x.reshape(N, C, HW).astype(jnp.bfloat16)
    y_flat = pl.pallas_call(
        body,
        out_shape=jax.ShapeDtypeStruct((N, O, HW), jnp.float32),
        grid=(N // B,),
        in_specs=[
            pl.BlockSpec((1, 1), lambda n: (0, 0)),
            pl.BlockSpec((B, C, HW), lambda n: (n, 0, 0)),
            pl.BlockSpec((T, O, C), lambda n: (0, 0, 0)),
            pl.BlockSpec((O, 1), lambda n: (0, 0)),
        ],
        out_specs=pl.BlockSpec((B, O, HW), lambda n: (n, 0, 0)),
        scratch_shapes=[
            pltpu.VMEM((C, padw), jnp.bfloat16),
        ],
        compiler_params=pltpu.CompilerParams(
            dimension_semantics=("parallel",)),
    )(inv_sig, xb, w_taps, bias.reshape(O, 1).astype(jnp.float32))

    return (y_flat.reshape(N, O, H, W), u_new.reshape(O), v_new.reshape(K))
